# Initial kernel scaffold; baseline (speedup 1.0000x reference)
import jax
import jax.numpy as jnp
from jax.experimental import pallas as pl


def kernel(x, Wr, br, W1, b1, W2, b2):
    raise NotImplementedError("write your pallas kernel here")


# speedup vs baseline: 1.2144x; 1.2144x over previous
import jax
import jax.numpy as jnp
from jax.experimental import pallas as pl
from jax.experimental.pallas import tpu as pltpu

T = 2048
IN_F = 2048
OUT_F = 8192
BLK = 512
NB = OUT_F // BLK
TOPK = NB // 4

BT = 1024


def _ffn_body(prob_ref, x_ref, w1_ref, b1_ref, w2_ref, b2_ref, y_ref):
    n = pl.program_id(1)
    prob = prob_ref[...]
    ids = jax.lax.broadcasted_iota(jnp.int32, (BT, NB), 1)
    pn = jnp.sum(jnp.where(ids == n, prob, 0.0), axis=1, keepdims=True)
    beats = (prob > pn) | ((prob == pn) & (ids < n))
    cnt = jnp.sum(beats.astype(jnp.int32), axis=1, keepdims=True)
    m = (cnt < TOPK).astype(jnp.float32)

    x = x_ref[...]
    w1 = w1_ref[...]
    h = jax.lax.dot_general(x, w1, (((1,), (1,)), ((), ())),
                            preferred_element_type=jnp.float32)
    h = h + b1_ref[0]
    g = jax.nn.gelu(h) * m
    w2 = w2_ref[...]
    contrib = jax.lax.dot_general(g, w2, (((1,), (1,)), ((), ())),
                                  preferred_element_type=jnp.float32)

    @pl.when(n == 0)
    def _init():
        y_ref[...] = b2_ref[...] + contrib

    @pl.when(n > 0)
    def _acc():
        y_ref[...] += contrib


def kernel(x, Wr, br, W1, b1, W2, b2):
    logits = x @ Wr.T + br[None, :]
    prob = jax.nn.softmax(logits, axis=-1)

    b1r = b1.reshape(NB, 1, BLK)
    b2r = b2.reshape(1, IN_F)

    y = pl.pallas_call(
        _ffn_body,
        grid=(T // BT, NB),
        in_specs=[
            pl.BlockSpec((BT, NB), lambda t, n: (t, 0)),
            pl.BlockSpec((BT, IN_F), lambda t, n: (t, 0)),
            pl.BlockSpec((BLK, IN_F), lambda t, n: (n, 0)),
            pl.BlockSpec((1, 1, BLK), lambda t, n: (n, 0, 0)),
            pl.BlockSpec((IN_F, BLK), lambda t, n: (0, n)),
            pl.BlockSpec((1, IN_F), lambda t, n: (0, 0)),
        ],
        out_specs=pl.BlockSpec((BT, IN_F), lambda t, n: (t, 0)),
        out_shape=jax.ShapeDtypeStruct((T, IN_F), jnp.float32),
        compiler_params=pltpu.CompilerParams(
            dimension_semantics=("parallel", "arbitrary"),
        ),
    )(prob, x, W1, b1r, W2, b2r)
    return y
